# baseline (speedup 1.0000x reference)
import jax
import jax.numpy as jnp
from jax import lax
from jax.experimental import pallas as pl
from jax.experimental.pallas import tpu as pltpu
from jax.experimental.pallas import tpu_sc as plsc

_B = 16384
_D = 32
_NC = 2
_NS = 16
_NW = _NC * _NS
_BPW = _B // _NW
_BLK = 16
_RING = 8


def _gather_concat_kernel(u_idx, i_idx, c_idx, s_idx,
                          w_u, w_i, w_c, w_s,
                          out,
                          idx_vs, blocks, rows, sems):
    wid = lax.axis_index("s") * _NC + lax.axis_index("c")
    base = wid * _BPW
    idx_refs = (u_idx, i_idx, c_idx, s_idx)
    tab_refs = (w_u, w_i, w_c, w_s)

    for f in range(4):
        pltpu.sync_copy(idx_refs[f].at[pl.ds(base, _BPW)], idx_vs[f])

    lane_iota = lax.iota(jnp.int32, 16)
    g_lo = lane_iota // 8
    s_all = lane_iota % 8

    def extract(v, slot, b, f):
        l_vec = jnp.full((16,), v % 128, jnp.int32)
        slot_vec = jnp.full((16,), slot, jnp.int32)
        for h in range(2):
            x = plsc.load_gather(
                blocks, [slot_vec, g_lo + 2 * h, s_all, l_vec])
            rows[b, pl.ds(f * _D + h * 16, 16)] = x

    def step(i, _):
        for f in range(4):
            vec = idx_vs[f][pl.ds(i * _BLK, _BLK)]
            vs = []
            pend = []
            for j in range(_BLK):
                v = lax.reduce_sum_p.bind(
                    jnp.where(lane_iota == j, vec, 0), axes=(0,))
                vs.append(v)
                slot = j % _RING
                if len(pend) == _RING:
                    cp0, (v0, b0, j0) = pend.pop(0)
                    cp0.wait()
                    extract(v0, j0 % _RING, b0, f)
                c128 = pl.multiple_of((v // 128) * 128, 128)
                b = i * _BLK + j
                cp = pltpu.async_copy(
                    tab_refs[f].at[:, :, pl.ds(c128, 128)],
                    blocks.at[slot], sems[slot])
                pend.append((cp, (v, b, j)))
            while pend:
                cp, (v0, b0, j0) = pend.pop(0)
                cp.wait()
                extract(v0, j0 % _RING, b0, f)
        return ()

    lax.fori_loop(0, _BPW // _BLK, step, ())
    pltpu.sync_copy(rows, out.at[pl.ds(base, _BPW), :])


@jax.jit
def _run(user_id, item_id, category, shop_id, W_user, W_item, W_category, W_shop):
    mesh = plsc.VectorSubcoreMesh(core_axis_name="c", subcore_axis_name="s")
    return pl.kernel(
        _gather_concat_kernel,
        out_type=jax.ShapeDtypeStruct((_B, 4 * _D), jnp.float32),
        mesh=mesh,
        compiler_params=pltpu.CompilerParams(needs_layout_passes=False),
        scratch_types=[
            [pltpu.VMEM((_BPW,), jnp.int32)] * 4,
            pltpu.VMEM((_RING, 4, 8, 128), jnp.float32),
            pltpu.VMEM((_BPW, 4 * _D), jnp.float32),
            [pltpu.SemaphoreType.DMA] * _RING,
        ],
    )(user_id, item_id, category, shop_id,
      W_user.T.reshape(4, 8, W_user.shape[0]),
      W_item.T.reshape(4, 8, W_item.shape[0]),
      W_category.T.reshape(4, 8, W_category.shape[0]),
      W_shop.T.reshape(4, 8, W_shop.shape[0]))


def kernel(user_id, item_id, category, shop_id, W_user, W_item, W_category, W_shop):
    return _run(user_id, item_id, category, shop_id,
                W_user, W_item, W_category, W_shop)


# speedup vs baseline: 1.1180x; 1.1180x over previous
import jax
import jax.numpy as jnp
from jax import lax
from jax.experimental import pallas as pl
from jax.experimental.pallas import tpu as pltpu
from jax.experimental.pallas import tpu_sc as plsc

_B = 16384
_D = 32
_NC = 2
_NS = 16
_NW = _NC * _NS
_BPW = _B // _NW
_BLK = 16
_RING = 8


def _gather_concat_kernel(u_idx, i_idx, c_idx, s_idx,
                          w_u, w_i, w_c, w_s,
                          out,
                          idx_vs, blocks, rows, sems):
    wid = lax.axis_index("s") * _NC + lax.axis_index("c")
    base = wid * _BPW
    idx_refs = (u_idx, i_idx, c_idx, s_idx)
    tab_refs = (w_u, w_i, w_c, w_s)

    for f in range(4):
        pltpu.sync_copy(idx_refs[f].at[pl.ds(base, _BPW)], idx_vs[f])

    lane_iota = lax.iota(jnp.int32, 16)
    g_lo = lane_iota // 8
    s_all = lane_iota % 8

    def extract(v, slot, b, f):
        l_vec = jnp.full((16,), v % 128, jnp.int32)
        slot_vec = jnp.full((16,), slot, jnp.int32)
        for h in range(2):
            x = plsc.load_gather(
                blocks, [slot_vec, g_lo + 2 * h, s_all, l_vec])
            rows[b, pl.ds(f * _D + h * 16, 16)] = x

    def step(i, _):
        pend = []
        n = 0
        for f in range(4):
            vec = idx_vs[f][pl.ds(i * _BLK, _BLK)]
            for j in range(_BLK):
                v = lax.reduce_sum_p.bind(
                    jnp.where(lane_iota == j, vec, 0), axes=(0,))
                slot = n % _RING
                if len(pend) == _RING:
                    cp0, (v0, b0, s0, f0) = pend.pop(0)
                    cp0.wait()
                    extract(v0, s0, b0, f0)
                c128 = pl.multiple_of((v // 128) * 128, 128)
                b = i * _BLK + j
                cp = pltpu.async_copy(
                    tab_refs[f].at[:, :, pl.ds(c128, 128)],
                    blocks.at[slot], sems[slot])
                pend.append((cp, (v, b, slot, f)))
                n += 1
        while pend:
            cp, (v0, b0, s0, f0) = pend.pop(0)
            cp.wait()
            extract(v0, s0, b0, f0)
        return ()

    lax.fori_loop(0, _BPW // _BLK, step, ())
    pltpu.sync_copy(rows, out.at[pl.ds(base, _BPW), :])


@jax.jit
def _run(user_id, item_id, category, shop_id, W_user, W_item, W_category, W_shop):
    mesh = plsc.VectorSubcoreMesh(core_axis_name="c", subcore_axis_name="s")
    return pl.kernel(
        _gather_concat_kernel,
        out_type=jax.ShapeDtypeStruct((_B, 4 * _D), jnp.float32),
        mesh=mesh,
        compiler_params=pltpu.CompilerParams(needs_layout_passes=False),
        scratch_types=[
            [pltpu.VMEM((_BPW,), jnp.int32)] * 4,
            pltpu.VMEM((_RING, 4, 8, 128), jnp.float32),
            pltpu.VMEM((_BPW, 4 * _D), jnp.float32),
            [pltpu.SemaphoreType.DMA] * _RING,
        ],
    )(user_id, item_id, category, shop_id,
      W_user.T.reshape(4, 8, W_user.shape[0]),
      W_item.T.reshape(4, 8, W_item.shape[0]),
      W_category.T.reshape(4, 8, W_category.shape[0]),
      W_shop.T.reshape(4, 8, W_shop.shape[0]))


def kernel(user_id, item_id, category, shop_id, W_user, W_item, W_category, W_shop):
    return _run(user_id, item_id, category, shop_id,
                W_user, W_item, W_category, W_shop)


# speedup vs baseline: 1.1308x; 1.0115x over previous
import jax
import jax.numpy as jnp
from jax import lax
from jax.experimental import pallas as pl
from jax.experimental.pallas import tpu as pltpu
from jax.experimental.pallas import tpu_sc as plsc

_B = 16384
_D = 32
_NC = 2
_NS = 16
_NW = _NC * _NS
_BPW = _B // _NW
_BLK = 32
_RING = 12


def _gather_concat_kernel(u_idx, i_idx, c_idx, s_idx,
                          w_u, w_i, w_c, w_s,
                          out,
                          idx_vs, blocks, rows, sems):
    wid = lax.axis_index("s") * _NC + lax.axis_index("c")
    base = wid * _BPW
    idx_refs = (u_idx, i_idx, c_idx, s_idx)
    tab_refs = (w_u, w_i, w_c, w_s)

    for f in range(4):
        pltpu.sync_copy(idx_refs[f].at[pl.ds(base, _BPW)], idx_vs[f])

    lane_iota = lax.iota(jnp.int32, 16)
    g_lo = lane_iota // 8
    s_all = lane_iota % 8

    def extract(v, slot, b, f):
        l_vec = jnp.full((16,), v % 128, jnp.int32)
        slot_vec = jnp.full((16,), slot, jnp.int32)
        for h in range(2):
            x = plsc.load_gather(
                blocks, [slot_vec, g_lo + 2 * h, s_all, l_vec])
            rows[b, pl.ds(f * _D + h * 16, 16)] = x

    def step(i, _):
        pend = []
        n = 0
        for f in range(4):
            for j in range(_BLK):
                if j % 16 == 0:
                    vec = idx_vs[f][pl.ds(i * _BLK + j, 16)]
                v = lax.reduce_sum_p.bind(
                    jnp.where(lane_iota == (j % 16), vec, 0), axes=(0,))
                slot = n % _RING
                if len(pend) == _RING:
                    cp0, (v0, b0, s0, f0) = pend.pop(0)
                    cp0.wait()
                    extract(v0, s0, b0, f0)
                c128 = pl.multiple_of((v // 128) * 128, 128)
                b = i * _BLK + j
                cp = pltpu.async_copy(
                    tab_refs[f].at[:, :, pl.ds(c128, 128)],
                    blocks.at[slot], sems[slot])
                pend.append((cp, (v, b, slot, f)))
                n += 1
        while pend:
            cp, (v0, b0, s0, f0) = pend.pop(0)
            cp.wait()
            extract(v0, s0, b0, f0)
        return ()

    lax.fori_loop(0, _BPW // _BLK, step, ())
    pltpu.sync_copy(rows, out.at[pl.ds(base, _BPW), :])


@jax.jit
def _run(user_id, item_id, category, shop_id, W_user, W_item, W_category, W_shop):
    mesh = plsc.VectorSubcoreMesh(core_axis_name="c", subcore_axis_name="s")
    return pl.kernel(
        _gather_concat_kernel,
        out_type=jax.ShapeDtypeStruct((_B, 4 * _D), jnp.float32),
        mesh=mesh,
        compiler_params=pltpu.CompilerParams(needs_layout_passes=False),
        scratch_types=[
            [pltpu.VMEM((_BPW,), jnp.int32)] * 4,
            pltpu.VMEM((_RING, 4, 8, 128), jnp.float32),
            pltpu.VMEM((_BPW, 4 * _D), jnp.float32),
            [pltpu.SemaphoreType.DMA] * _RING,
        ],
    )(user_id, item_id, category, shop_id,
      W_user.T.reshape(4, 8, W_user.shape[0]),
      W_item.T.reshape(4, 8, W_item.shape[0]),
      W_category.T.reshape(4, 8, W_category.shape[0]),
      W_shop.T.reshape(4, 8, W_shop.shape[0]))


def kernel(user_id, item_id, category, shop_id, W_user, W_item, W_category, W_shop):
    return _run(user_id, item_id, category, shop_id,
                W_user, W_item, W_category, W_shop)
